# baseline (speedup 1.0000x reference)
import functools

import jax
import jax.numpy as jnp
from jax import lax
from jax.experimental import pallas as pl
from jax.experimental.pallas import tpu as pltpu
from jax.experimental.pallas import tpu_sc as plsc

N_NODES = 10000
N_PAD = 10240
N_EDGES = 320000
D = 128
B = 64
NS = 16
NC = 2
EPW = N_EDGES // NS
P0_CH = 4000
P_CH = 8000
VOCAB32 = 32
SEG_W = 65


def _seg_body(bnds_ref, out_ref):
    ids = (lax.broadcasted_iota(jnp.int32, (N_PAD // 128, 128), 0) * 128
           + lax.broadcasted_iota(jnp.int32, (N_PAD // 128, 128), 1))
    seg = jnp.zeros((N_PAD // 128, 128), jnp.int32)
    for p in range(B):
        seg += (ids >= bnds_ref[0, p]).astype(jnp.int32)
    out_ref[...] = seg


def _compute_seg(bnds):
    return pl.pallas_call(
        _seg_body,
        out_shape=jax.ShapeDtypeStruct((N_PAD // 128, 128), jnp.int32),
        in_specs=[pl.BlockSpec(memory_space=pltpu.SMEM)],
    )(bnds.reshape(1, B)).reshape(N_PAD)


def _sc_body(src_hbm, dst_hbm, x_hbm, seg_hbm,
             s1_out, c2_out, hx_out, kb1, kb2,
             x_v, seg_v, cnt_col, s1_col, c2_col, hx_col,
             s1_x, c2_x, hx_x,
             eb_s, eb_d, k1_b, k2_b, kb_a, kb_b, sem_a, sem_b):
    c = lax.axis_index("c")
    s = lax.axis_index("s")
    wid = c * NS + s
    pltpu.sync_copy(x_hbm, x_v)
    pltpu.sync_copy(seg_hbm, seg_v)

    zf = jnp.zeros((16,), jnp.float32)

    def _zero(i, _):
        cnt_col[pl.ds(i * 16, 16)] = zf
        return 0
    lax.fori_loop(0, N_PAD // 16, _zero, 0, unroll=8)
    for col in (s1_col, c2_col, hx_col):
        for i in range(8):
            col[pl.ds(i * 16, 16)] = zf

    e0 = s * EPW
    for ch in range(EPW // P0_CH):
        base = e0 + ch * P0_CH
        pltpu.sync_copy(src_hbm.at[pl.ds(base, P0_CH)], eb_s)
        pltpu.sync_copy(dst_hbm.at[pl.ds(base, P0_CH)], eb_d)

        def _p0(j, _):
            sv = eb_s[pl.ds(j * 16, 16)]
            dv = eb_d[pl.ds(j * 16, 16)]
            xv = plsc.load_gather(x_v, [sv])
            sg = plsc.load_gather(seg_v, [dv])
            k1_b[pl.ds(j * 16, 16)] = dv * VOCAB32 + xv
            k2_b[pl.ds(j * 16, 16)] = sv * 128 + sg
            return 0
        lax.fori_loop(0, P0_CH // 16, _p0, 0, unroll=5)
        pltpu.sync_copy(k1_b, kb1.at[pl.ds(base, P0_CH)])
        pltpu.sync_copy(k2_b, kb2.at[pl.ds(base, P0_CH)])
    plsc.subcore_barrier()

    onesf = jnp.ones((16,), jnp.float32)
    n_ch = N_EDGES // P_CH

    def _stream(kb_ref, process):
        pltpu.async_copy(kb_ref.at[pl.ds(0, P_CH)], kb_a, sem_a)

        def _pair(g, _):
            ch0 = g * 2
            pltpu.make_async_copy(kb_ref.at[pl.ds(0, P_CH)], kb_a, sem_a).wait()
            base1 = pl.multiple_of((ch0 + 1) * P_CH, 8)
            pltpu.async_copy(kb_ref.at[pl.ds(base1, P_CH)], kb_b, sem_b)
            process(kb_a)
            pltpu.make_async_copy(kb_ref.at[pl.ds(0, P_CH)], kb_b, sem_b).wait()
            nxt = jnp.minimum(ch0 + 2, n_ch - 1)
            base2 = pl.multiple_of(nxt * P_CH, 8)
            pltpu.async_copy(kb_ref.at[pl.ds(base2, P_CH)], kb_a, sem_a)
            process(kb_b)
            return 0
        lax.fori_loop(0, n_ch // 2, _pair, 0)
        pltpu.make_async_copy(kb_ref.at[pl.ds(0, P_CH)], kb_a, sem_a).wait()

    @pl.when(wid < 26)
    def _():
        v = wid
        lane_sw = lax.iota(jnp.int32, 16) * SEG_W

        def _zero_x(i, _):
            s1_x[pl.ds(i * 16, 16)] = zf
            c2_x[pl.ds(i * 16, 16)] = zf
            hx_x[pl.ds(i * 16, 16)] = zf
            return 0
        lax.fori_loop(0, 16 * SEG_W // 16, _zero_x, 0, unroll=8)

        def _p1(buf):
            def _b(j, _):
                k = buf[pl.ds(j * 16, 16)]
                dv = lax.shift_right_logical(k, 5)
                xv = k & (VOCAB32 - 1)
                plsc.addupdate_scatter(cnt_col, [dv], onesf, mask=xv == v)
                return 0
            lax.fori_loop(0, P_CH // 16, _b, 0, unroll=10)
        _stream(kb1, _p1)

        def _nodes(i, _):
            cv = cnt_col[pl.ds(i * 16, 16)]
            sg = seg_v[pl.ds(i * 16, 16)]
            xb = x_v[pl.ds(i * 16, 16)]
            idx = lane_sw + sg
            plsc.addupdate_scatter(c2_x, [idx], cv)
            plsc.addupdate_scatter(hx_x, [idx], onesf, mask=xb == v)
            return 0
        lax.fori_loop(0, N_NODES // 16, _nodes, 0, unroll=5)

        def _p2(buf):
            def _b(j, _):
                k = buf[pl.ds(j * 16, 16)]
                sv = lax.shift_right_logical(k, 7)
                sg = k & 127
                cval = plsc.load_gather(cnt_col, [sv])
                plsc.addupdate_scatter(s1_x, [lane_sw + sg], cval)
                return 0
            lax.fori_loop(0, P_CH // 16, _b, 0, unroll=10)
        _stream(kb2, _p2)

        for j in range(SEG_W // 16):
            a_s1 = zf
            a_c2 = zf
            a_hx = zf
            for l in range(16):
                off = l * SEG_W + j * 16
                a_s1 = a_s1 + s1_x[pl.ds(off, 16)]
                a_c2 = a_c2 + c2_x[pl.ds(off, 16)]
                a_hx = a_hx + hx_x[pl.ds(off, 16)]
            s1_col[pl.ds(j * 16, 16)] = a_s1
            c2_col[pl.ds(j * 16, 16)] = a_c2
            hx_col[pl.ds(j * 16, 16)] = a_hx

    pltpu.sync_copy(s1_col, s1_out.at[wid, 0])
    pltpu.sync_copy(c2_col, c2_out.at[wid, 0])
    pltpu.sync_copy(hx_col, hx_out.at[wid, 0])


def _sc_counts(src, dst, xp, seg):
    mesh = plsc.VectorSubcoreMesh(core_axis_name="c", subcore_axis_name="s",
                                  num_cores=NC, num_subcores=NS)
    f = pl.kernel(
        _sc_body,
        out_type=[
            jax.ShapeDtypeStruct((32, 1, 128), jnp.float32),
            jax.ShapeDtypeStruct((32, 1, 128), jnp.float32),
            jax.ShapeDtypeStruct((32, 1, 128), jnp.float32),
            jax.ShapeDtypeStruct((N_EDGES,), jnp.int32),
            jax.ShapeDtypeStruct((N_EDGES,), jnp.int32),
        ],
        mesh=mesh,
        compiler_params=pltpu.CompilerParams(needs_layout_passes=False),
        scratch_types=[
            pltpu.VMEM((N_PAD,), jnp.int32),
            pltpu.VMEM((N_PAD,), jnp.int32),
            pltpu.VMEM((N_PAD,), jnp.float32),
            pltpu.VMEM((128,), jnp.float32),
            pltpu.VMEM((128,), jnp.float32),
            pltpu.VMEM((128,), jnp.float32),
            pltpu.VMEM((16 * SEG_W,), jnp.float32),
            pltpu.VMEM((16 * SEG_W,), jnp.float32),
            pltpu.VMEM((16 * SEG_W,), jnp.float32),
            pltpu.VMEM((P0_CH,), jnp.int32),
            pltpu.VMEM((P0_CH,), jnp.int32),
            pltpu.VMEM((P0_CH,), jnp.int32),
            pltpu.VMEM((P0_CH,), jnp.int32),
            pltpu.VMEM((P_CH,), jnp.int32),
            pltpu.VMEM((P_CH,), jnp.int32),
            pltpu.SemaphoreType.DMA,
            pltpu.SemaphoreType.DMA,
        ],
    )
    return f(src, dst, xp, seg)


def _dot_nn(a, b):
    return lax.dot_general(a, b, (((1,), (0,)), ((), ())),
                           preferred_element_type=jnp.float32,
                           precision=lax.Precision.HIGHEST)


def _dot_nt(a, b):
    return lax.dot_general(a, b, (((1,), (1,)), ((), ())),
                           preferred_element_type=jnp.float32,
                           precision=lax.Precision.HIGHEST)


def _fin_body(s1, c2, hx, embp, wr1, wo1, wr2, wo2, br1, br2, cn, out):
    A = _dot_nt(embp[...], wr1[...])
    R = _dot_nt(embp[...], wo1[...])
    C2 = c2[...]
    E = jnp.sum(C2, axis=1, keepdims=True)
    n = cn[...]
    G = _dot_nn(s1[...], A) + E * br1[...] + _dot_nn(C2, R)
    H = _dot_nn(C2, A) + n * br1[...] + _dot_nn(hx[...], R)
    out[...] = (_dot_nt(G, wr2[...]) + n * br2[...] + _dot_nt(H, wo2[...])) / n


def _finalize(S1, C2, Hx, embP, W_rel1, W_root1, W_rel2, W_root2,
              b_rel1, b_rel2, counts):
    return pl.pallas_call(
        _fin_body,
        out_shape=jax.ShapeDtypeStruct((B, D), jnp.float32),
    )(S1, C2, Hx, embP, W_rel1, W_root1, W_rel2, W_root2,
      b_rel1, b_rel2, counts)


def kernel(x, edge_index, amino_acids_numbers, emb,
           W_rel1, b_rel1, W_root1, W_rel2, b_rel2, W_root2):
    x = x.astype(jnp.int32)
    src = edge_index[0].astype(jnp.int32)
    dst = edge_index[1].astype(jnp.int32)
    bnds = amino_acids_numbers.astype(jnp.int32)

    seg = _compute_seg(bnds)
    xp = jnp.concatenate([x, jnp.zeros((N_PAD - N_NODES,), jnp.int32)])
    s1t, c2t, hxt, _, _ = _sc_counts(src, dst, xp, seg)

    S1 = s1t.reshape(32, 128).T[:B]
    C2 = c2t.reshape(32, 128).T[:B]
    Hx = hxt.reshape(32, 128).T[:B]
    embP = jnp.zeros((VOCAB32, D), jnp.float32).at[:emb.shape[0]].set(emb)
    starts = jnp.concatenate([jnp.zeros((1,), bnds.dtype), bnds[:-1]])
    counts = (bnds - starts).astype(jnp.float32).reshape(B, 1)
    return _finalize(S1, C2, Hx, embP, W_rel1, W_root1, W_rel2, W_root2,
                     b_rel1.reshape(1, D), b_rel2.reshape(1, D), counts)


# speedup vs baseline: 2.7410x; 2.7410x over previous
import functools

import jax
import jax.numpy as jnp
from jax import lax
from jax.experimental import pallas as pl
from jax.experimental.pallas import tpu as pltpu
from jax.experimental.pallas import tpu_sc as plsc

N_NODES = 10000
N_PAD = 10240
N_EDGES = 320000
D = 128
B = 64
NS = 16
NC = 2
EPW = N_EDGES // NS
P0_CH = 4000
P_CH = 8000
VOCAB32 = 32
SEG_W = 65


def _seg_body(bnds_ref, out_ref):
    ids = (lax.broadcasted_iota(jnp.int32, (N_PAD // 128, 128), 0) * 128
           + lax.broadcasted_iota(jnp.int32, (N_PAD // 128, 128), 1))
    seg = jnp.zeros((N_PAD // 128, 128), jnp.int32)
    for p in range(B):
        seg += (ids >= bnds_ref[0, p]).astype(jnp.int32)
    out_ref[...] = seg


def _compute_seg(bnds):
    return pl.pallas_call(
        _seg_body,
        out_shape=jax.ShapeDtypeStruct((N_PAD // 128, 128), jnp.int32),
        in_specs=[pl.BlockSpec(memory_space=pltpu.SMEM)],
    )(bnds.reshape(1, B)).reshape(N_PAD)


def _sc_body(src_hbm, dst_hbm, x_hbm, seg_hbm,
             s1_out, c2_out, hx_out, kb1, kb2,
             x_v, seg_v, cnt_col, s1_col, c2_col, hx_col,
             s1_x, c2_x, hx_x,
             eb_s, eb_d, k1_b, k2_b, kb_a, kb_b, sem_a, sem_b):
    c = lax.axis_index("c")
    s = lax.axis_index("s")
    wid = c * NS + s
    pltpu.sync_copy(x_hbm, x_v)
    pltpu.sync_copy(seg_hbm, seg_v)

    zf = jnp.zeros((16,), jnp.float32)

    @plsc.parallel_loop(0, N_PAD // 16, unroll=8)
    def _zero(i):
        cnt_col[pl.ds(i * 16, 16)] = zf
    for col in (s1_col, c2_col, hx_col):
        for i in range(8):
            col[pl.ds(i * 16, 16)] = zf

    e0 = s * EPW
    for ch in range(EPW // P0_CH):
        base = e0 + ch * P0_CH
        pltpu.sync_copy(src_hbm.at[pl.ds(base, P0_CH)], eb_s)
        pltpu.sync_copy(dst_hbm.at[pl.ds(base, P0_CH)], eb_d)

        @plsc.parallel_loop(0, P0_CH // 16, unroll=5)
        def _p0(j):
            sv = eb_s[pl.ds(j * 16, 16)]
            dv = eb_d[pl.ds(j * 16, 16)]
            xv = plsc.load_gather(x_v, [sv])
            sg = plsc.load_gather(seg_v, [dv])
            k1_b[pl.ds(j * 16, 16)] = dv * VOCAB32 + xv
            k2_b[pl.ds(j * 16, 16)] = sv * 128 + sg
        pltpu.sync_copy(k1_b, kb1.at[pl.ds(base, P0_CH)])
        pltpu.sync_copy(k2_b, kb2.at[pl.ds(base, P0_CH)])
    plsc.subcore_barrier()

    onesf = jnp.ones((16,), jnp.float32)
    n_ch = N_EDGES // P_CH

    def _stream(kb_ref, process):
        pltpu.async_copy(kb_ref.at[pl.ds(0, P_CH)], kb_a, sem_a)

        def _pair(g, _):
            ch0 = g * 2
            pltpu.make_async_copy(kb_ref.at[pl.ds(0, P_CH)], kb_a, sem_a).wait()
            base1 = pl.multiple_of((ch0 + 1) * P_CH, 8)
            pltpu.async_copy(kb_ref.at[pl.ds(base1, P_CH)], kb_b, sem_b)
            process(kb_a)
            pltpu.make_async_copy(kb_ref.at[pl.ds(0, P_CH)], kb_b, sem_b).wait()
            nxt = jnp.minimum(ch0 + 2, n_ch - 1)
            base2 = pl.multiple_of(nxt * P_CH, 8)
            pltpu.async_copy(kb_ref.at[pl.ds(base2, P_CH)], kb_a, sem_a)
            process(kb_b)
            return 0
        lax.fori_loop(0, n_ch // 2, _pair, 0)
        pltpu.make_async_copy(kb_ref.at[pl.ds(0, P_CH)], kb_a, sem_a).wait()

    @pl.when(wid < 26)
    def _():
        v = wid
        lane_sw = lax.iota(jnp.int32, 16) * SEG_W

        @plsc.parallel_loop(0, 16 * SEG_W // 16, unroll=8)
        def _zero_x(i):
            s1_x[pl.ds(i * 16, 16)] = zf
            c2_x[pl.ds(i * 16, 16)] = zf
            hx_x[pl.ds(i * 16, 16)] = zf

        def _p1(buf):
            @plsc.parallel_loop(0, P_CH // 16, unroll=8)
            def _b(j):
                k = buf[pl.ds(j * 16, 16)]
                dv = lax.shift_right_logical(k, 5)
                xv = k & (VOCAB32 - 1)
                plsc.addupdate_scatter(cnt_col, [dv], onesf, mask=xv == v)
        _stream(kb1, _p1)

        @plsc.parallel_loop(0, N_NODES // 16, unroll=5)
        def _nodes(i):
            cv = cnt_col[pl.ds(i * 16, 16)]
            sg = seg_v[pl.ds(i * 16, 16)]
            xb = x_v[pl.ds(i * 16, 16)]
            idx = lane_sw + sg
            plsc.addupdate_scatter(c2_x, [idx], cv)
            plsc.addupdate_scatter(hx_x, [idx], onesf, mask=xb == v)

        def _p2(buf):
            @plsc.parallel_loop(0, P_CH // 16, unroll=8)
            def _b(j):
                k = buf[pl.ds(j * 16, 16)]
                sv = lax.shift_right_logical(k, 7)
                sg = k & 127
                cval = plsc.load_gather(cnt_col, [sv])
                plsc.addupdate_scatter(s1_x, [lane_sw + sg], cval)
        _stream(kb2, _p2)

        for j in range(SEG_W // 16):
            a_s1 = zf
            a_c2 = zf
            a_hx = zf
            for l in range(16):
                off = l * SEG_W + j * 16
                a_s1 = a_s1 + s1_x[pl.ds(off, 16)]
                a_c2 = a_c2 + c2_x[pl.ds(off, 16)]
                a_hx = a_hx + hx_x[pl.ds(off, 16)]
            s1_col[pl.ds(j * 16, 16)] = a_s1
            c2_col[pl.ds(j * 16, 16)] = a_c2
            hx_col[pl.ds(j * 16, 16)] = a_hx

    pltpu.sync_copy(s1_col, s1_out.at[wid, 0])
    pltpu.sync_copy(c2_col, c2_out.at[wid, 0])
    pltpu.sync_copy(hx_col, hx_out.at[wid, 0])


def _sc_counts(src, dst, xp, seg):
    mesh = plsc.VectorSubcoreMesh(core_axis_name="c", subcore_axis_name="s",
                                  num_cores=NC, num_subcores=NS)
    f = pl.kernel(
        _sc_body,
        out_type=[
            jax.ShapeDtypeStruct((32, 1, 128), jnp.float32),
            jax.ShapeDtypeStruct((32, 1, 128), jnp.float32),
            jax.ShapeDtypeStruct((32, 1, 128), jnp.float32),
            jax.ShapeDtypeStruct((N_EDGES,), jnp.int32),
            jax.ShapeDtypeStruct((N_EDGES,), jnp.int32),
        ],
        mesh=mesh,
        compiler_params=pltpu.CompilerParams(needs_layout_passes=False),
        scratch_types=[
            pltpu.VMEM((N_PAD,), jnp.int32),
            pltpu.VMEM((N_PAD,), jnp.int32),
            pltpu.VMEM((N_PAD,), jnp.float32),
            pltpu.VMEM((128,), jnp.float32),
            pltpu.VMEM((128,), jnp.float32),
            pltpu.VMEM((128,), jnp.float32),
            pltpu.VMEM((16 * SEG_W,), jnp.float32),
            pltpu.VMEM((16 * SEG_W,), jnp.float32),
            pltpu.VMEM((16 * SEG_W,), jnp.float32),
            pltpu.VMEM((P0_CH,), jnp.int32),
            pltpu.VMEM((P0_CH,), jnp.int32),
            pltpu.VMEM((P0_CH,), jnp.int32),
            pltpu.VMEM((P0_CH,), jnp.int32),
            pltpu.VMEM((P_CH,), jnp.int32),
            pltpu.VMEM((P_CH,), jnp.int32),
            pltpu.SemaphoreType.DMA,
            pltpu.SemaphoreType.DMA,
        ],
    )
    return f(src, dst, xp, seg)


def _dot_nn(a, b):
    return lax.dot_general(a, b, (((1,), (0,)), ((), ())),
                           preferred_element_type=jnp.float32,
                           precision=lax.Precision.HIGHEST)


def _dot_nt(a, b):
    return lax.dot_general(a, b, (((1,), (1,)), ((), ())),
                           preferred_element_type=jnp.float32,
                           precision=lax.Precision.HIGHEST)


def _fin_body(s1, c2, hx, embp, wr1, wo1, wr2, wo2, br1, br2, cn, out):
    A = _dot_nt(embp[...], wr1[...])
    R = _dot_nt(embp[...], wo1[...])
    C2 = c2[...]
    E = jnp.sum(C2, axis=1, keepdims=True)
    n = cn[...]
    G = _dot_nn(s1[...], A) + E * br1[...] + _dot_nn(C2, R)
    H = _dot_nn(C2, A) + n * br1[...] + _dot_nn(hx[...], R)
    out[...] = (_dot_nt(G, wr2[...]) + n * br2[...] + _dot_nt(H, wo2[...])) / n


def _finalize(S1, C2, Hx, embP, W_rel1, W_root1, W_rel2, W_root2,
              b_rel1, b_rel2, counts):
    return pl.pallas_call(
        _fin_body,
        out_shape=jax.ShapeDtypeStruct((B, D), jnp.float32),
    )(S1, C2, Hx, embP, W_rel1, W_root1, W_rel2, W_root2,
      b_rel1, b_rel2, counts)


def kernel(x, edge_index, amino_acids_numbers, emb,
           W_rel1, b_rel1, W_root1, W_rel2, b_rel2, W_root2):
    x = x.astype(jnp.int32)
    src = edge_index[0].astype(jnp.int32)
    dst = edge_index[1].astype(jnp.int32)
    bnds = amino_acids_numbers.astype(jnp.int32)

    seg = _compute_seg(bnds)
    xp = jnp.concatenate([x, jnp.zeros((N_PAD - N_NODES,), jnp.int32)])
    s1t, c2t, hxt, _, _ = _sc_counts(src, dst, xp, seg)

    S1 = s1t.reshape(32, 128).T[:B]
    C2 = c2t.reshape(32, 128).T[:B]
    Hx = hxt.reshape(32, 128).T[:B]
    embP = jnp.zeros((VOCAB32, D), jnp.float32).at[:emb.shape[0]].set(emb)
    starts = jnp.concatenate([jnp.zeros((1,), bnds.dtype), bnds[:-1]])
    counts = (bnds - starts).astype(jnp.float32).reshape(B, 1)
    return _finalize(S1, C2, Hx, embP, W_rel1, W_root1, W_rel2, W_root2,
                     b_rel1.reshape(1, D), b_rel2.reshape(1, D), counts)


# speedup vs baseline: 2.7521x; 1.0040x over previous
import functools

import jax
import jax.numpy as jnp
from jax import lax
from jax.experimental import pallas as pl
from jax.experimental.pallas import tpu as pltpu
from jax.experimental.pallas import tpu_sc as plsc

N_NODES = 10000
N_PAD = 10240
N_EDGES = 320000
D = 128
B = 64
NS = 16
NC = 2
EPW = N_EDGES // NS
P0_CH = 4000
P_CH = 8000
VOCAB32 = 32
SEG_W = 65


def _seg_body(bnds_ref, out_ref):
    ids = (lax.broadcasted_iota(jnp.int32, (N_PAD // 128, 128), 0) * 128
           + lax.broadcasted_iota(jnp.int32, (N_PAD // 128, 128), 1))
    seg = jnp.zeros((N_PAD // 128, 128), jnp.int32)
    for p in range(B):
        seg += (ids >= bnds_ref[0, p]).astype(jnp.int32)
    out_ref[...] = seg


def _compute_seg(bnds):
    return pl.pallas_call(
        _seg_body,
        out_shape=jax.ShapeDtypeStruct((N_PAD // 128, 128), jnp.int32),
        in_specs=[pl.BlockSpec(memory_space=pltpu.SMEM)],
    )(bnds.reshape(1, B)).reshape(N_PAD)


def _sc_body(src_hbm, dst_hbm, x_hbm, seg_hbm,
             s1_out, c2_out, hx_out, kb1, kb2,
             x_v, seg_v, cnt_col, s1_col, c2_col, hx_col,
             s1_x, c2_x, hx_x,
             eb_s, eb_d, k1_b, k2_b, kb_a, kb_b, sem_a, sem_b):
    c = lax.axis_index("c")
    s = lax.axis_index("s")
    wid = c * NS + s
    pltpu.sync_copy(x_hbm, x_v)
    pltpu.sync_copy(seg_hbm, seg_v)

    zf = jnp.zeros((16,), jnp.float32)

    @plsc.parallel_loop(0, N_PAD // 16, unroll=8)
    def _zero(i):
        cnt_col[pl.ds(i * 16, 16)] = zf
    for col in (s1_col, c2_col, hx_col):
        for i in range(8):
            col[pl.ds(i * 16, 16)] = zf

    e0 = s * EPW
    for ch in range(EPW // P0_CH):
        base = e0 + ch * P0_CH
        pltpu.sync_copy(src_hbm.at[pl.ds(base, P0_CH)], eb_s)
        pltpu.sync_copy(dst_hbm.at[pl.ds(base, P0_CH)], eb_d)

        @plsc.parallel_loop(0, P0_CH // 16, unroll=5)
        def _p0(j):
            sv = eb_s[pl.ds(j * 16, 16)]
            dv = eb_d[pl.ds(j * 16, 16)]
            xv = plsc.load_gather(x_v, [sv])
            sg = plsc.load_gather(seg_v, [dv])
            k1_b[pl.ds(j * 16, 16)] = dv * VOCAB32 + xv
            k2_b[pl.ds(j * 16, 16)] = sv * 128 + sg
        pltpu.sync_copy(k1_b, kb1.at[pl.ds(base, P0_CH)])
        pltpu.sync_copy(k2_b, kb2.at[pl.ds(base, P0_CH)])
    plsc.subcore_barrier()

    onesf = jnp.ones((16,), jnp.float32)
    n_ch = N_EDGES // P_CH

    def _stream(kb_ref, process):
        pltpu.async_copy(kb_ref.at[pl.ds(0, P_CH)], kb_a, sem_a)

        def _pair(g, _):
            ch0 = g * 2
            pltpu.make_async_copy(kb_ref.at[pl.ds(0, P_CH)], kb_a, sem_a).wait()
            base1 = pl.multiple_of((ch0 + 1) * P_CH, 8)
            pltpu.async_copy(kb_ref.at[pl.ds(base1, P_CH)], kb_b, sem_b)
            process(kb_a)
            pltpu.make_async_copy(kb_ref.at[pl.ds(0, P_CH)], kb_b, sem_b).wait()
            nxt = jnp.minimum(ch0 + 2, n_ch - 1)
            base2 = pl.multiple_of(nxt * P_CH, 8)
            pltpu.async_copy(kb_ref.at[pl.ds(base2, P_CH)], kb_a, sem_a)
            process(kb_b)
            return 0
        lax.fori_loop(0, n_ch // 2, _pair, 0)
        pltpu.make_async_copy(kb_ref.at[pl.ds(0, P_CH)], kb_a, sem_a).wait()

    @pl.when(wid < 26)
    def _():
        v = wid
        lane_sw = lax.iota(jnp.int32, 16) * SEG_W

        @plsc.parallel_loop(0, 16 * SEG_W // 16, unroll=8)
        def _zero_x(i):
            s1_x[pl.ds(i * 16, 16)] = zf
            c2_x[pl.ds(i * 16, 16)] = zf
            hx_x[pl.ds(i * 16, 16)] = zf

        def _p1(buf):
            @plsc.parallel_loop(0, P_CH // 16, unroll=16)
            def _b(j):
                k = buf[pl.ds(j * 16, 16)]
                dv = lax.shift_right_logical(k, 5)
                xv = k & (VOCAB32 - 1)
                plsc.addupdate_scatter(cnt_col, [dv], onesf, mask=xv == v)
        _stream(kb1, _p1)

        @plsc.parallel_loop(0, N_NODES // 16, unroll=5)
        def _nodes(i):
            cv = cnt_col[pl.ds(i * 16, 16)]
            sg = seg_v[pl.ds(i * 16, 16)]
            xb = x_v[pl.ds(i * 16, 16)]
            idx = lane_sw + sg
            plsc.addupdate_scatter(c2_x, [idx], cv)
            plsc.addupdate_scatter(hx_x, [idx], onesf, mask=xb == v)

        def _p2(buf):
            @plsc.parallel_loop(0, P_CH // 16, unroll=16)
            def _b(j):
                k = buf[pl.ds(j * 16, 16)]
                sv = lax.shift_right_logical(k, 7)
                sg = k & 127
                cval = plsc.load_gather(cnt_col, [sv])
                plsc.addupdate_scatter(s1_x, [lane_sw + sg], cval)
        _stream(kb2, _p2)

        for j in range(SEG_W // 16):
            a_s1 = zf
            a_c2 = zf
            a_hx = zf
            for l in range(16):
                off = l * SEG_W + j * 16
                a_s1 = a_s1 + s1_x[pl.ds(off, 16)]
                a_c2 = a_c2 + c2_x[pl.ds(off, 16)]
                a_hx = a_hx + hx_x[pl.ds(off, 16)]
            s1_col[pl.ds(j * 16, 16)] = a_s1
            c2_col[pl.ds(j * 16, 16)] = a_c2
            hx_col[pl.ds(j * 16, 16)] = a_hx

    pltpu.sync_copy(s1_col, s1_out.at[wid, 0])
    pltpu.sync_copy(c2_col, c2_out.at[wid, 0])
    pltpu.sync_copy(hx_col, hx_out.at[wid, 0])


def _sc_counts(src, dst, xp, seg):
    mesh = plsc.VectorSubcoreMesh(core_axis_name="c", subcore_axis_name="s",
                                  num_cores=NC, num_subcores=NS)
    f = pl.kernel(
        _sc_body,
        out_type=[
            jax.ShapeDtypeStruct((32, 1, 128), jnp.float32),
            jax.ShapeDtypeStruct((32, 1, 128), jnp.float32),
            jax.ShapeDtypeStruct((32, 1, 128), jnp.float32),
            jax.ShapeDtypeStruct((N_EDGES,), jnp.int32),
            jax.ShapeDtypeStruct((N_EDGES,), jnp.int32),
        ],
        mesh=mesh,
        compiler_params=pltpu.CompilerParams(needs_layout_passes=False),
        scratch_types=[
            pltpu.VMEM((N_PAD,), jnp.int32),
            pltpu.VMEM((N_PAD,), jnp.int32),
            pltpu.VMEM((N_PAD,), jnp.float32),
            pltpu.VMEM((128,), jnp.float32),
            pltpu.VMEM((128,), jnp.float32),
            pltpu.VMEM((128,), jnp.float32),
            pltpu.VMEM((16 * SEG_W,), jnp.float32),
            pltpu.VMEM((16 * SEG_W,), jnp.float32),
            pltpu.VMEM((16 * SEG_W,), jnp.float32),
            pltpu.VMEM((P0_CH,), jnp.int32),
            pltpu.VMEM((P0_CH,), jnp.int32),
            pltpu.VMEM((P0_CH,), jnp.int32),
            pltpu.VMEM((P0_CH,), jnp.int32),
            pltpu.VMEM((P_CH,), jnp.int32),
            pltpu.VMEM((P_CH,), jnp.int32),
            pltpu.SemaphoreType.DMA,
            pltpu.SemaphoreType.DMA,
        ],
    )
    return f(src, dst, xp, seg)


def _dot_nn(a, b):
    return lax.dot_general(a, b, (((1,), (0,)), ((), ())),
                           preferred_element_type=jnp.float32,
                           precision=lax.Precision.HIGHEST)


def _dot_nt(a, b):
    return lax.dot_general(a, b, (((1,), (1,)), ((), ())),
                           preferred_element_type=jnp.float32,
                           precision=lax.Precision.HIGHEST)


def _fin_body(s1, c2, hx, embp, wr1, wo1, wr2, wo2, br1, br2, cn, out):
    A = _dot_nt(embp[...], wr1[...])
    R = _dot_nt(embp[...], wo1[...])
    C2 = c2[...]
    E = jnp.sum(C2, axis=1, keepdims=True)
    n = cn[...]
    G = _dot_nn(s1[...], A) + E * br1[...] + _dot_nn(C2, R)
    H = _dot_nn(C2, A) + n * br1[...] + _dot_nn(hx[...], R)
    out[...] = (_dot_nt(G, wr2[...]) + n * br2[...] + _dot_nt(H, wo2[...])) / n


def _finalize(S1, C2, Hx, embP, W_rel1, W_root1, W_rel2, W_root2,
              b_rel1, b_rel2, counts):
    return pl.pallas_call(
        _fin_body,
        out_shape=jax.ShapeDtypeStruct((B, D), jnp.float32),
    )(S1, C2, Hx, embP, W_rel1, W_root1, W_rel2, W_root2,
      b_rel1, b_rel2, counts)


def kernel(x, edge_index, amino_acids_numbers, emb,
           W_rel1, b_rel1, W_root1, W_rel2, b_rel2, W_root2):
    x = x.astype(jnp.int32)
    src = edge_index[0].astype(jnp.int32)
    dst = edge_index[1].astype(jnp.int32)
    bnds = amino_acids_numbers.astype(jnp.int32)

    seg = _compute_seg(bnds)
    xp = jnp.concatenate([x, jnp.zeros((N_PAD - N_NODES,), jnp.int32)])
    s1t, c2t, hxt, _, _ = _sc_counts(src, dst, xp, seg)

    S1 = s1t.reshape(32, 128).T[:B]
    C2 = c2t.reshape(32, 128).T[:B]
    Hx = hxt.reshape(32, 128).T[:B]
    embP = jnp.zeros((VOCAB32, D), jnp.float32).at[:emb.shape[0]].set(emb)
    starts = jnp.concatenate([jnp.zeros((1,), bnds.dtype), bnds[:-1]])
    counts = (bnds - starts).astype(jnp.float32).reshape(B, 1)
    return _finalize(S1, C2, Hx, embP, W_rel1, W_root1, W_rel2, W_root2,
                     b_rel1.reshape(1, D), b_rel2.reshape(1, D), counts)
